# Initial kernel scaffold; baseline (speedup 1.0000x reference)
import jax
import jax.numpy as jnp
from jax.experimental import pallas as pl


def kernel(x_feature, train_label, centers):
    raise NotImplementedError("write your pallas kernel here")


# speedup vs baseline: 2.3105x; 2.3105x over previous
import functools

import jax
import jax.numpy as jnp
from jax.experimental import pallas as pl
from jax.experimental.pallas import tpu as pltpu

_TARGET_CLASS = 1


def _fused_body(lab_ref, x_ref, c_ref, out_ref, acc_s_ref, acc_c_ref):
    i = pl.program_id(0)
    nblk = pl.num_programs(0)

    @pl.when(i == 0)
    def _init():
        acc_s_ref[...] = jnp.zeros_like(acc_s_ref)
        acc_c_ref[...] = jnp.zeros_like(acc_c_ref)

    x = x_ref[...]
    c = c_ref[...]
    v = jax.lax.dot_general(
        x, c, (((1,), (1,)), ((), ())),
        preferred_element_type=jnp.float32)
    inv_n = jax.lax.rsqrt(jnp.maximum(jnp.sum(c * c, axis=1), 1e-24))
    s = v * inv_n[None, :]
    ms = jnp.max(s, axis=1, keepdims=True)
    iota = jax.lax.broadcasted_iota(jnp.int32, s.shape, 1)
    big = jnp.int32(2**30)
    idx = jnp.min(jnp.where(s >= ms, iota, big), axis=1, keepdims=True)
    best_v = jnp.sum(jnp.where(iota == idx, v, 0.0), axis=1, keepdims=True)
    m = jnp.max(v, axis=1, keepdims=True)
    lse = m + jnp.log(jnp.sum(jnp.exp(v - m), axis=1, keepdims=True))
    nll = lse - best_v
    lab = lab_ref[0, 0, :]
    maskf = (lab == _TARGET_CLASS).astype(jnp.float32)[:, None]
    acc_s_ref[...] += nll * maskf
    acc_c_ref[...] += maskf

    @pl.when(i == nblk - 1)
    def _fin():
        tot_s = jnp.sum(acc_s_ref[...], axis=(0, 1), keepdims=True)
        tot_c = jnp.sum(acc_c_ref[...], axis=(0, 1), keepdims=True)
        out_ref[...] = tot_s / tot_c


@functools.partial(jax.jit, static_argnames=("bn", "interpret"))
def _multi_center_loss(x_feature, train_label, centers, bn=256, interpret=False):
    n, d = x_feature.shape
    c, _ = centers.shape
    nblk = n // bn
    lab3 = train_label.astype(jnp.int32).reshape(nblk, 1, bn)
    out = pl.pallas_call(
        _fused_body,
        grid=(nblk,),
        in_specs=[
            pl.BlockSpec((1, 1, bn), lambda i: (i, 0, 0)),
            pl.BlockSpec((bn, d), lambda i: (i, 0)),
            pl.BlockSpec((c, d), lambda i: (0, 0)),
        ],
        out_specs=pl.BlockSpec((1, 1), lambda i: (0, 0)),
        out_shape=jax.ShapeDtypeStruct((1, 1), jnp.float32),
        scratch_shapes=[
            pltpu.VMEM((bn, 1), jnp.float32),
            pltpu.VMEM((bn, 1), jnp.float32),
        ],
        compiler_params=pltpu.CompilerParams(
            dimension_semantics=("arbitrary",)),
        interpret=interpret,
    )(lab3, x_feature, centers)
    return out[0, 0]


def kernel(x_feature, train_label, centers):
    return _multi_center_loss(x_feature, train_label, centers)


# speedup vs baseline: 11.4998x; 4.9772x over previous
import functools

import jax
import jax.numpy as jnp
from jax import lax
from jax.experimental import pallas as pl
from jax.experimental.pallas import tpu as pltpu
from jax.experimental.pallas import tpu_sc as plsc

_TARGET_CLASS = 1

_NC = 2
_NS = 16
_NW = _NC * _NS
_CH = 128
_BN = 256


def _make_sc_compact(n, d):
    seg_len = n // _NW
    ngrp = seg_len // 16
    mesh = plsc.VectorSubcoreMesh(core_axis_name="c", subcore_axis_name="s")

    @functools.partial(
        pl.kernel,
        out_type=[
            jax.ShapeDtypeStruct((n, d), jnp.float32),
            jax.ShapeDtypeStruct((_NW, 16), jnp.int32),
        ],
        mesh=mesh,
        compiler_params=pltpu.CompilerParams(needs_layout_passes=False),
        scratch_types=[
            pltpu.VMEM((seg_len,), jnp.int32),
            pltpu.VMEM((seg_len + 16,), jnp.int32),
            pltpu.VMEM((_CH, d), jnp.float32),
            pltpu.VMEM((16,), jnp.int32),
            pltpu.VMEM((16,), jnp.int32),
            pltpu.SemaphoreType.DMA,
        ],
    )
    def sc_compact(x_hbm, lab_hbm, xc_hbm, cnt_hbm, lab_v, cidx_v, rows_v,
                   cbuf_v, spill_v, sem):
        wid = lax.axis_index("s") * _NC + lax.axis_index("c")
        seg = wid * seg_len
        pltpu.sync_copy(lab_hbm.at[pl.ds(seg, seg_len)], lab_v)
        iota = lax.iota(jnp.int32, 16)

        def prefill(g, carry):
            cidx_v[pl.ds(g * 16, 16)] = jnp.broadcast_to(seg, (16,))
            return carry

        lax.fori_loop(0, ngrp + 1, prefill, 0)

        tgt16 = jnp.full((16,), _TARGET_CLASS, jnp.int32)
        sixteen16 = jnp.full((16,), 16, jnp.int32)

        def compact(g, cnt):
            lab16 = lab_v[pl.ds(g * 16, 16)]
            m = lab16 == tgt16
            idx16 = jnp.broadcast_to(seg + g * 16, (16,)) + iota
            key = jnp.where(m, iota, iota + sixteen16)
            _, vals = plsc.sort_key_val(key, idx16)
            cidx_v[pl.ds(cnt, 16)] = vals
            pc = plsc.all_reduce_population_count(m)
            return cnt + pc[0]

        cnt = lax.fori_loop(0, ngrp, compact, jnp.int32(0))

        nch = (cnt + _CH - 1) // _CH

        def chunk(k, carry):
            pltpu.async_copy(
                x_hbm.at[cidx_v.at[pl.ds(k * _CH, _CH)]], rows_v, sem).wait()
            pltpu.sync_copy(rows_v, xc_hbm.at[pl.ds(seg + k * _CH, _CH)])
            return carry

        lax.fori_loop(0, nch, chunk, 0)

        cbuf_v[...] = jnp.broadcast_to(cnt, (16,))
        pltpu.sync_copy(cbuf_v, cnt_hbm.at[wid])

    return sc_compact


def _tc_body(cnt_ref, x_ref, c_ref, out_ref, acc_ref, *, bn, nw, d):
    w = pl.program_id(0)
    j = pl.program_id(1)
    nj = pl.num_programs(1)

    @pl.when((w == 0) & (j == 0))
    def _init():
        acc_ref[...] = jnp.zeros_like(acc_ref)

    cnt_w = cnt_ref[w]

    @pl.when(j * bn < cnt_w)
    def _compute():
        x = x_ref[...][:, :d]
        c = c_ref[...]
        v = lax.dot_general(
            x, c, (((1,), (1,)), ((), ())),
            preferred_element_type=jnp.float32)
        inv_n = lax.rsqrt(jnp.maximum(jnp.sum(c * c, axis=1), 1e-24))
        s = v * inv_n[None, :]
        ms = jnp.max(s, axis=1, keepdims=True)
        iota_c = lax.broadcasted_iota(jnp.int32, s.shape, 1)
        big = jnp.int32(2**30)
        idx = jnp.min(jnp.where(s >= ms, iota_c, big), axis=1, keepdims=True)
        best_v = jnp.sum(jnp.where(iota_c == idx, v, 0.0), axis=1,
                         keepdims=True)
        m = jnp.max(v, axis=1, keepdims=True)
        lse = m + jnp.log(jnp.sum(jnp.exp(v - m), axis=1, keepdims=True))
        nll = lse - best_v
        row = lax.broadcasted_iota(jnp.int32, (bn, 1), 0)
        valid = (j * bn + row) < cnt_w
        acc_ref[...] += jnp.where(valid, nll, 0.0)

    @pl.when((w == nw - 1) & (j == nj - 1))
    def _fin():
        total = jnp.int32(0)
        for ww in range(nw):
            total = total + cnt_ref[ww]
        tot_s = jnp.sum(acc_ref[...], axis=(0, 1), keepdims=True)
        out_ref[...] = tot_s / total.astype(jnp.float32)


def _tc_loss(xc, counts, centers, *, nw, bn):
    n, dpad = xc.shape
    c, d = centers.shape
    seg_len = n // nw
    nj = seg_len // bn

    def xmap(w, j, cnt_ref):
        nb = jnp.maximum((cnt_ref[w] + bn - 1) // bn, 1)
        return (w * nj + jnp.minimum(j, nb - 1), 0)

    grid_spec = pltpu.PrefetchScalarGridSpec(
        num_scalar_prefetch=1,
        grid=(nw, nj),
        in_specs=[
            pl.BlockSpec((bn, dpad), xmap),
            pl.BlockSpec((c, d), lambda w, j, cnt_ref: (0, 0)),
        ],
        out_specs=pl.BlockSpec((1, 1), lambda w, j, cnt_ref: (0, 0)),
        scratch_shapes=[pltpu.VMEM((bn, 1), jnp.float32)],
    )
    out = pl.pallas_call(
        functools.partial(_tc_body, bn=bn, nw=nw, d=d),
        grid_spec=grid_spec,
        out_shape=jax.ShapeDtypeStruct((1, 1), jnp.float32),
        compiler_params=pltpu.CompilerParams(
            dimension_semantics=("arbitrary", "arbitrary")),
    )(counts, xc, centers)
    return out[0, 0]


@jax.jit
def _multi_center_loss(x_feature, train_label, centers):
    n, d = x_feature.shape
    dpad = 128
    xp = jnp.pad(x_feature, ((0, 0), (0, dpad - d)))
    xc, counts2d = _make_sc_compact(n, dpad)(
        xp, train_label.astype(jnp.int32))
    return _tc_loss(xc, counts2d[:, 0], centers, nw=_NW, bn=_BN)


def kernel(x_feature, train_label, centers):
    return _multi_center_loss(x_feature, train_label, centers)


# speedup vs baseline: 13.7789x; 1.1982x over previous
import functools

import jax
import jax.numpy as jnp
from jax import lax
from jax.experimental import pallas as pl
from jax.experimental.pallas import tpu as pltpu
from jax.experimental.pallas import tpu_sc as plsc

_TARGET_CLASS = 1

_NC = 2
_NS = 16
_NW = _NC * _NS
_CH = 128
_BN = 256


def _make_sc_compact(n, d):
    seg_len = n // _NW
    ngrp = seg_len // 16
    mesh = plsc.VectorSubcoreMesh(core_axis_name="c", subcore_axis_name="s")

    @functools.partial(
        pl.kernel,
        out_type=[
            jax.ShapeDtypeStruct((n, d), jnp.float32),
            jax.ShapeDtypeStruct((_NW, 16), jnp.int32),
        ],
        mesh=mesh,
        compiler_params=pltpu.CompilerParams(needs_layout_passes=False),
        scratch_types=[
            pltpu.VMEM((seg_len,), jnp.int32),
            pltpu.VMEM((seg_len + 16,), jnp.int32),
            pltpu.VMEM((_CH, d), jnp.float32),
            pltpu.VMEM((16,), jnp.int32),
            pltpu.VMEM((16,), jnp.int32),
            pltpu.SemaphoreType.DMA,
        ],
    )
    def sc_compact(x_hbm, lab_hbm, xc_hbm, cnt_hbm, lab_v, cidx_v, rows_v,
                   cbuf_v, spill_v, sem):
        wid = lax.axis_index("s") * _NC + lax.axis_index("c")
        seg = wid * seg_len
        pltpu.sync_copy(lab_hbm.at[pl.ds(seg, seg_len)], lab_v)
        iota = lax.iota(jnp.int32, 16)

        def prefill(g, carry):
            cidx_v[pl.ds(g * 16, 16)] = jnp.broadcast_to(seg, (16,))
            return carry

        lax.fori_loop(0, ngrp + 1, prefill, 0)

        tgt16 = jnp.full((16,), _TARGET_CLASS, jnp.int32)
        sixteen16 = jnp.full((16,), 16, jnp.int32)

        def compact(g, cnt):
            lab16 = lab_v[pl.ds(g * 16, 16)]
            m = lab16 == tgt16
            idx16 = jnp.broadcast_to(seg + g * 16, (16,)) + iota
            key = jnp.where(m, iota, iota + sixteen16)
            _, vals = plsc.sort_key_val(key, idx16)
            cidx_v[pl.ds(cnt, 16)] = vals
            pc = plsc.all_reduce_population_count(m)
            return cnt + pc[0]

        cnt = lax.fori_loop(0, ngrp, compact, jnp.int32(0))

        nch = (cnt + _CH - 1) // _CH

        def chunk(k, carry):
            pltpu.async_copy(
                x_hbm.at[cidx_v.at[pl.ds(k * _CH, _CH)]], rows_v, sem).wait()
            pltpu.sync_copy(rows_v, xc_hbm.at[pl.ds(seg + k * _CH, _CH)])
            return carry

        lax.fori_loop(0, nch, chunk, 0)

        cbuf_v[...] = jnp.broadcast_to(cnt, (16,))
        pltpu.sync_copy(cbuf_v, cnt_hbm.at[wid])

    return sc_compact


def _tc_body(cnt_ref, x_ref, c_ref, out_ref, acc_ref, cs_ref, *, bn, nw, d):
    w = pl.program_id(0)
    j = pl.program_id(1)
    nj = pl.num_programs(1)

    @pl.when((w == 0) & (j == 0))
    def _init():
        acc_ref[...] = jnp.zeros_like(acc_ref)
        c = c_ref[...]
        inv_n = lax.rsqrt(jnp.maximum(
            jnp.sum(c * c, axis=1, keepdims=True), 1e-24))
        cs_ref[...] = c * inv_n

    cnt_w = cnt_ref[w]

    @pl.when(j * bn < cnt_w)
    def _compute():
        x = x_ref[...][:, :d]
        c = c_ref[...]
        v = lax.dot_general(
            x, c, (((1,), (1,)), ((), ())),
            preferred_element_type=jnp.float32)
        s = lax.dot_general(
            x, cs_ref[...], (((1,), (1,)), ((), ())),
            preferred_element_type=jnp.float32)
        ms = jnp.max(s, axis=1, keepdims=True)
        best_v = jnp.max(jnp.where(s >= ms, v, -jnp.inf), axis=1,
                         keepdims=True)
        m = jnp.max(v, axis=1, keepdims=True)
        lse = m + jnp.log(jnp.sum(jnp.exp(v - m), axis=1, keepdims=True))
        nll = lse - best_v
        row = lax.broadcasted_iota(jnp.int32, (bn, 1), 0)
        valid = (j * bn + row) < cnt_w
        acc_ref[...] += jnp.where(valid, nll, 0.0)

    @pl.when((w == nw - 1) & (j == nj - 1))
    def _fin():
        total = jnp.int32(0)
        for ww in range(nw):
            total = total + cnt_ref[ww]
        tot_s = jnp.sum(acc_ref[...], axis=(0, 1), keepdims=True)
        out_ref[...] = tot_s / total.astype(jnp.float32)


def _tc_loss(xc, counts, centers, *, nw, bn):
    n, dpad = xc.shape
    c, d = centers.shape
    seg_len = n // nw
    nj = seg_len // bn

    def xmap(w, j, cnt_ref):
        nb = jnp.maximum((cnt_ref[w] + bn - 1) // bn, 1)
        return (w * nj + jnp.minimum(j, nb - 1), 0)

    grid_spec = pltpu.PrefetchScalarGridSpec(
        num_scalar_prefetch=1,
        grid=(nw, nj),
        in_specs=[
            pl.BlockSpec((bn, dpad), xmap),
            pl.BlockSpec((c, d), lambda w, j, cnt_ref: (0, 0)),
        ],
        out_specs=pl.BlockSpec((1, 1), lambda w, j, cnt_ref: (0, 0)),
        scratch_shapes=[
            pltpu.VMEM((bn, 1), jnp.float32),
            pltpu.VMEM((c, d), jnp.float32),
        ],
    )
    out = pl.pallas_call(
        functools.partial(_tc_body, bn=bn, nw=nw, d=d),
        grid_spec=grid_spec,
        out_shape=jax.ShapeDtypeStruct((1, 1), jnp.float32),
        compiler_params=pltpu.CompilerParams(
            dimension_semantics=("arbitrary", "arbitrary")),
    )(counts, xc, centers)
    return out[0, 0]


@jax.jit
def _multi_center_loss(x_feature, train_label, centers):
    n, d = x_feature.shape
    dpad = 128
    xp = jnp.pad(x_feature, ((0, 0), (0, dpad - d)))
    xc, counts2d = _make_sc_compact(n, dpad)(
        xp, train_label.astype(jnp.int32))
    return _tc_loss(xc, counts2d[:, 0], centers, nw=_NW, bn=_BN)


def kernel(x_feature, train_label, centers):
    return _multi_center_loss(x_feature, train_label, centers)


# speedup vs baseline: 16.4389x; 1.1931x over previous
import functools

import jax
import jax.numpy as jnp
from jax import lax
from jax.experimental import pallas as pl
from jax.experimental.pallas import tpu as pltpu
from jax.experimental.pallas import tpu_sc as plsc

_TARGET_CLASS = 1

_NC = 2
_NS = 16
_NW = _NC * _NS
_CH = 128
_BN = 256


def _make_sc_compact(n, d):
    seg_len = n // _NW
    ngrp = seg_len // 16
    mesh = plsc.VectorSubcoreMesh(core_axis_name="c", subcore_axis_name="s")

    @functools.partial(
        pl.kernel,
        out_type=[
            jax.ShapeDtypeStruct((n, d), jnp.float32),
            jax.ShapeDtypeStruct((_NW, 16), jnp.int32),
        ],
        mesh=mesh,
        compiler_params=pltpu.CompilerParams(needs_layout_passes=False),
        scratch_types=[
            pltpu.VMEM((seg_len,), jnp.int32),
            pltpu.VMEM((seg_len + 16,), jnp.int32),
            pltpu.VMEM((_CH, d), jnp.float32),
            pltpu.VMEM((16,), jnp.int32),
            pltpu.VMEM((16,), jnp.int32),
            pltpu.SemaphoreType.DMA,
        ],
    )
    def sc_compact(x_hbm, lab_hbm, xc_hbm, cnt_hbm, lab_v, cidx_v, rows_v,
                   cbuf_v, spill_v, sem):
        wid = lax.axis_index("s") * _NC + lax.axis_index("c")
        seg = wid * seg_len
        pltpu.sync_copy(lab_hbm.at[pl.ds(seg, seg_len)], lab_v)
        iota = lax.iota(jnp.int32, 16)

        def prefill(g, carry):
            cidx_v[pl.ds(g * 16, 16)] = jnp.broadcast_to(seg, (16,))
            return carry

        lax.fori_loop(0, ngrp + 1, prefill, 0)

        tgt16 = jnp.full((16,), _TARGET_CLASS, jnp.int32)
        sixteen16 = jnp.full((16,), 16, jnp.int32)

        def compact(g, cnt):
            lab16 = lab_v[pl.ds(g * 16, 16)]
            m = lab16 == tgt16
            idx16 = jnp.broadcast_to(seg + g * 16, (16,)) + iota
            key = jnp.where(m, iota, iota + sixteen16)
            _, vals = plsc.sort_key_val(key, idx16)
            cidx_v[pl.ds(cnt, 16)] = vals
            pc = plsc.all_reduce_population_count(m)
            return cnt + pc[0]

        cnt = lax.fori_loop(0, ngrp, compact, jnp.int32(0))

        nch = (cnt + _CH - 1) // _CH

        def chunk(k, carry):
            pltpu.async_copy(
                x_hbm.at[cidx_v.at[pl.ds(k * _CH, _CH)]], rows_v, sem).wait()
            pltpu.sync_copy(rows_v, xc_hbm.at[pl.ds(seg + k * _CH, _CH)])
            return carry

        lax.fori_loop(0, nch, chunk, 0)

        cbuf_v[...] = jnp.broadcast_to(cnt, (16,))
        pltpu.sync_copy(cbuf_v, cnt_hbm.at[wid])

    return sc_compact


def _tc_body(cnt_ref, x_ref, c_ref, out_ref, acc_ref, cs_ref, *, bn, nw, d):
    w = pl.program_id(0)
    j = pl.program_id(1)
    nj = pl.num_programs(1)

    @pl.when((w == 0) & (j == 0))
    def _init():
        acc_ref[...] = jnp.zeros_like(acc_ref)
        c = c_ref[...]
        inv_n = lax.rsqrt(jnp.maximum(
            jnp.sum(c * c, axis=1, keepdims=True), 1e-24))
        cs_ref[...] = c * inv_n

    cnt_w = cnt_ref[w]

    @pl.when(j * bn < cnt_w)
    def _compute():
        x = x_ref[...][:, :d]
        c = c_ref[...]
        v = lax.dot_general(
            x, c, (((1,), (1,)), ((), ())),
            preferred_element_type=jnp.float32)
        s = lax.dot_general(
            x, cs_ref[...], (((1,), (1,)), ((), ())),
            preferred_element_type=jnp.float32)
        ms = jnp.max(s, axis=1, keepdims=True)
        best_v = jnp.max(jnp.where(s >= ms, v, -jnp.inf), axis=1,
                         keepdims=True)
        m = jnp.max(v, axis=1, keepdims=True)
        lse = m + jnp.log(jnp.sum(jnp.exp(v - m), axis=1, keepdims=True))
        nll = lse - best_v
        row = lax.broadcasted_iota(jnp.int32, (bn, 1), 0)
        valid = (j * bn + row) < cnt_w
        acc_ref[...] += jnp.where(valid, nll, 0.0)

    @pl.when((w == nw - 1) & (j == nj - 1))
    def _fin():
        total = jnp.int32(0)
        for ww in range(nw):
            total = total + cnt_ref[ww]
        tot_s = jnp.sum(acc_ref[...], axis=(0, 1), keepdims=True)
        out_ref[...] = tot_s / total.astype(jnp.float32)


def _tc_loss(xc, counts, centers, *, nw, bn):
    n, dpad = xc.shape
    c, d = centers.shape
    seg_len = n // nw
    nj = seg_len // bn

    def xmap(w, j, cnt_ref):
        nb = jnp.maximum((cnt_ref[w] + bn - 1) // bn, 1)
        return (w * nj + jnp.minimum(j, nb - 1), 0)

    njd = jnp.maximum((jnp.max(counts) + bn - 1) // bn, 1).astype(jnp.int32)
    grid_spec = pltpu.PrefetchScalarGridSpec(
        num_scalar_prefetch=1,
        grid=(nw, njd),
        in_specs=[
            pl.BlockSpec((bn, dpad), xmap),
            pl.BlockSpec((c, d), lambda w, j, cnt_ref: (0, 0)),
        ],
        out_specs=pl.BlockSpec((1, 1), lambda w, j, cnt_ref: (0, 0)),
        scratch_shapes=[
            pltpu.VMEM((bn, 1), jnp.float32),
            pltpu.VMEM((c, d), jnp.float32),
        ],
    )
    out = pl.pallas_call(
        functools.partial(_tc_body, bn=bn, nw=nw, d=d),
        grid_spec=grid_spec,
        out_shape=jax.ShapeDtypeStruct((1, 1), jnp.float32),
        compiler_params=pltpu.CompilerParams(
            dimension_semantics=("arbitrary", "arbitrary")),
    )(counts, xc, centers)
    return out[0, 0]


@jax.jit
def _multi_center_loss(x_feature, train_label, centers):
    n, d = x_feature.shape
    dpad = 128
    xp = jnp.pad(x_feature, ((0, 0), (0, dpad - d)))
    xc, counts2d = _make_sc_compact(n, dpad)(
        xp, train_label.astype(jnp.int32))
    return _tc_loss(xc, counts2d[:, 0], centers, nw=_NW, bn=_BN)


def kernel(x_feature, train_label, centers):
    return _multi_center_loss(x_feature, train_label, centers)


# speedup vs baseline: 18.0882x; 1.1003x over previous
import functools

import jax
import jax.numpy as jnp
from jax import lax
from jax.experimental import pallas as pl
from jax.experimental.pallas import tpu as pltpu
from jax.experimental.pallas import tpu_sc as plsc

_TARGET_CLASS = 1

_NC = 2
_NS = 16
_NW = _NC * _NS
_CH = 128
_BN = 256


def _make_sc_compact(n, d):
    seg_len = n // _NW
    ngrp = seg_len // 16
    mesh = plsc.VectorSubcoreMesh(core_axis_name="c", subcore_axis_name="s")

    @functools.partial(
        pl.kernel,
        out_type=[
            jax.ShapeDtypeStruct((n, d), jnp.float32),
            jax.ShapeDtypeStruct((_NW, 16), jnp.int32),
        ],
        mesh=mesh,
        compiler_params=pltpu.CompilerParams(needs_layout_passes=False),
        scratch_types=[
            pltpu.VMEM((seg_len,), jnp.int32),
            pltpu.VMEM((seg_len + 16,), jnp.int32),
            pltpu.VMEM((_CH, d), jnp.float32),
            pltpu.VMEM((16,), jnp.int32),
            pltpu.VMEM((16,), jnp.int32),
            pltpu.SemaphoreType.DMA,
        ],
    )
    def sc_compact(x_hbm, lab_hbm, xc_hbm, cnt_hbm, lab_v, cidx_v, rows_v,
                   cbuf_v, spill_v, sem):
        wid = lax.axis_index("s") * _NC + lax.axis_index("c")
        seg = wid * seg_len
        pltpu.sync_copy(lab_hbm.at[pl.ds(seg, seg_len)], lab_v)
        iota = lax.iota(jnp.int32, 16)

        def prefill(g, carry):
            cidx_v[pl.ds(g * 16, 16)] = jnp.broadcast_to(seg, (16,))
            return carry

        lax.fori_loop(0, ngrp + 1, prefill, 0)

        tgt16 = jnp.full((16,), _TARGET_CLASS, jnp.int32)
        sixteen16 = jnp.full((16,), 16, jnp.int32)

        def compact(g, cnt):
            lab16 = lab_v[pl.ds(g * 16, 16)]
            m = lab16 == tgt16
            idx16 = jnp.broadcast_to(seg + g * 16, (16,)) + iota
            key = jnp.where(m, iota, iota + sixteen16)
            _, vals = plsc.sort_key_val(key, idx16)
            cidx_v[pl.ds(cnt, 16)] = vals
            pc = plsc.all_reduce_population_count(m)
            return cnt + pc[0]

        cnt = lax.fori_loop(0, ngrp, compact, jnp.int32(0))

        nch = (cnt + _CH - 1) // _CH

        def chunk(k, carry):
            pltpu.async_copy(
                x_hbm.at[cidx_v.at[pl.ds(k * _CH, _CH)]], rows_v, sem).wait()
            pltpu.sync_copy(rows_v, xc_hbm.at[pl.ds(seg + k * _CH, _CH)])
            return carry

        lax.fori_loop(0, nch, chunk, 0)

        cbuf_v[...] = jnp.broadcast_to(cnt, (16,))
        pltpu.sync_copy(cbuf_v, cnt_hbm.at[wid])

    return sc_compact


def _tc_body(cnt_ref, x_ref, c_ref, out_ref, acc_ref, inv_ref, *, bn, nw, d):
    w = pl.program_id(0)
    j = pl.program_id(1)
    nj = pl.num_programs(1)

    @pl.when((w == 0) & (j == 0))
    def _init():
        acc_ref[...] = jnp.zeros_like(acc_ref)
        c = c_ref[...]
        inv_n = lax.rsqrt(jnp.maximum(jnp.sum(c * c, axis=1), 1e-24))
        inv_ref[...] = inv_n[None, :]

    cnt_w = cnt_ref[w]

    @pl.when(j * bn < cnt_w)
    def _compute():
        x = x_ref[...][:, :d]
        c = c_ref[...]
        v = lax.dot_general(
            x, c, (((1,), (1,)), ((), ())),
            preferred_element_type=jnp.float32)
        s = v * inv_ref[...]
        ms = jnp.max(s, axis=1, keepdims=True)
        best_v = jnp.max(jnp.where(s >= ms, v, -jnp.inf), axis=1,
                         keepdims=True)
        m = jnp.max(v, axis=1, keepdims=True)
        lse = m + jnp.log(jnp.sum(jnp.exp(v - m), axis=1, keepdims=True))
        nll = lse - best_v
        row = lax.broadcasted_iota(jnp.int32, (bn, 1), 0)
        valid = (j * bn + row) < cnt_w
        acc_ref[...] += jnp.where(valid, nll, 0.0)

    @pl.when((w == nw - 1) & (j == nj - 1))
    def _fin():
        total = jnp.int32(0)
        for ww in range(nw):
            total = total + cnt_ref[ww]
        tot_s = jnp.sum(acc_ref[...], axis=(0, 1), keepdims=True)
        out_ref[...] = tot_s / total.astype(jnp.float32)


def _tc_loss(xc, counts, centers, *, nw, bn):
    n, dpad = xc.shape
    c, d = centers.shape
    seg_len = n // nw
    nj = seg_len // bn

    def xmap(w, j, cnt_ref):
        nb = jnp.maximum((cnt_ref[w] + bn - 1) // bn, 1)
        return (w * nj + jnp.minimum(j, nb - 1), 0)

    njd = jnp.maximum((jnp.max(counts) + bn - 1) // bn, 1).astype(jnp.int32)
    grid_spec = pltpu.PrefetchScalarGridSpec(
        num_scalar_prefetch=1,
        grid=(nw, njd),
        in_specs=[
            pl.BlockSpec((bn, dpad), xmap),
            pl.BlockSpec((c, d), lambda w, j, cnt_ref: (0, 0)),
        ],
        out_specs=pl.BlockSpec((1, 1), lambda w, j, cnt_ref: (0, 0)),
        scratch_shapes=[
            pltpu.VMEM((bn, 1), jnp.float32),
            pltpu.VMEM((1, c), jnp.float32),
        ],
    )
    out = pl.pallas_call(
        functools.partial(_tc_body, bn=bn, nw=nw, d=d),
        grid_spec=grid_spec,
        out_shape=jax.ShapeDtypeStruct((1, 1), jnp.float32),
        compiler_params=pltpu.CompilerParams(
            dimension_semantics=("arbitrary", "arbitrary")),
    )(counts, xc, centers)
    return out[0, 0]


@jax.jit
def _multi_center_loss(x_feature, train_label, centers):
    n, d = x_feature.shape
    dpad = 128
    xp = jnp.pad(x_feature, ((0, 0), (0, dpad - d)))
    xc, counts2d = _make_sc_compact(n, dpad)(
        xp, train_label.astype(jnp.int32))
    return _tc_loss(xc, counts2d[:, 0], centers, nw=_NW, bn=_BN)


def kernel(x_feature, train_label, centers):
    return _multi_center_loss(x_feature, train_label, centers)
